# Initial kernel scaffold; baseline (speedup 1.0000x reference)
import jax
import jax.numpy as jnp
from jax.experimental import pallas as pl


def kernel(word_table, pos1_table, pos2_table, word, pos1, pos2):
    raise NotImplementedError("write your pallas kernel here")


# speedup vs baseline: 4.6363x; 4.6363x over previous
import functools

import jax
import jax.numpy as jnp
from jax import lax
from jax.experimental import pallas as pl
from jax.experimental.pallas import tpu as pltpu
from jax.experimental.pallas import tpu_sc as plsc

B = 1024
L = 200
N = B * L
WORD_DIM = 128
POS_DIM = 16
OUT_D = WORD_DIM + 2 * POS_DIM

NC = 2
NS = 16
NW = NC * NS
CHUNK = 128
NCHUNKS = N // CHUNK
CPW = NCHUNKS // NW


def _lookup(word_table, pos1_table, pos2_table, wi, p1i, p2i):
    mesh = plsc.VectorSubcoreMesh(
        core_axis_name="c", subcore_axis_name="s", num_cores=NC, num_subcores=NS
    )

    @functools.partial(
        pl.kernel,
        out_type=jax.ShapeDtypeStruct((N, OUT_D), jnp.float32),
        mesh=mesh,
        compiler_params=pltpu.CompilerParams(use_tc_tiling_on_sc=False),
        scratch_types=[
            pltpu.VMEM((CPW, CHUNK), jnp.int32),
            pltpu.VMEM((CPW, CHUNK), jnp.int32),
            pltpu.VMEM((CPW, CHUNK), jnp.int32),
            pltpu.VMEM((CHUNK, WORD_DIM), jnp.float32),
            pltpu.VMEM((CHUNK, POS_DIM), jnp.float32),
            pltpu.VMEM((CHUNK, POS_DIM), jnp.float32),
            pltpu.SemaphoreType.DMA,
            pltpu.SemaphoreType.DMA,
            pltpu.SemaphoreType.DMA,
        ],
    )
    def k(wt, p1t, p2t, wi_h, p1i_h, p2i_h, out,
          widx_v, p1idx_v, p2idx_v, w_v, p1_v, p2_v, sem0, sem1, sem2):
        wid = lax.axis_index("s") * NC + lax.axis_index("c")
        cbase = wid * CPW
        pltpu.sync_copy(wi_h.at[wid], widx_v)
        pltpu.sync_copy(p1i_h.at[wid], p1idx_v)
        pltpu.sync_copy(p2i_h.at[wid], p2idx_v)

        def step(j, carry):
            cw = pltpu.async_copy(wt.at[widx_v.at[j]], w_v, sem0)
            c1 = pltpu.async_copy(p1t.at[p1idx_v.at[j]], p1_v, sem1)
            c2 = pltpu.async_copy(p2t.at[p2idx_v.at[j]], p2_v, sem2)
            cw.wait()
            c1.wait()
            c2.wait()
            row = (cbase + j) * CHUNK
            pltpu.sync_copy(w_v, out.at[pl.ds(row, CHUNK), pl.ds(0, WORD_DIM)])
            pltpu.sync_copy(p1_v, out.at[pl.ds(row, CHUNK), pl.ds(WORD_DIM, POS_DIM)])
            pltpu.sync_copy(p2_v, out.at[pl.ds(row, CHUNK), pl.ds(WORD_DIM + POS_DIM, POS_DIM)])
            return carry

        lax.fori_loop(0, CPW, step, 0)

    return k(word_table, pos1_table, pos2_table, wi, p1i, p2i)


def kernel(word_table, pos1_table, pos2_table, word, pos1, pos2):
    wi = word.reshape(N).astype(jnp.int32).reshape(NW, CPW, CHUNK)
    p1i = pos1.reshape(N).astype(jnp.int32).reshape(NW, CPW, CHUNK)
    p2i = pos2.reshape(N).astype(jnp.int32).reshape(NW, CPW, CHUNK)
    out = _lookup(word_table, pos1_table, pos2_table, wi, p1i, p2i)
    return out.reshape(B, L, OUT_D)


# speedup vs baseline: 4.6670x; 1.0066x over previous
import functools

import jax
import jax.numpy as jnp
from jax import lax
from jax.experimental import pallas as pl
from jax.experimental.pallas import tpu as pltpu
from jax.experimental.pallas import tpu_sc as plsc

B = 1024
L = 200
N = B * L
WORD_DIM = 128
POS_DIM = 16
OUT_D = WORD_DIM + 2 * POS_DIM

NC = 2
NS = 16
NW = NC * NS
CHUNK = 128
NCHUNKS = N // CHUNK
CPW = NCHUNKS // NW
NB = 5
NSTEP = CPW // NB


def _lookup(word_table, pos1_table, pos2_table, wi, p1i, p2i):
    mesh = plsc.VectorSubcoreMesh(
        core_axis_name="c", subcore_axis_name="s", num_cores=NC, num_subcores=NS
    )

    @functools.partial(
        pl.kernel,
        out_type=jax.ShapeDtypeStruct((N, OUT_D), jnp.float32),
        mesh=mesh,
        compiler_params=pltpu.CompilerParams(use_tc_tiling_on_sc=False),
        scratch_types=[
            pltpu.VMEM((CPW, CHUNK), jnp.int32),
            pltpu.VMEM((CPW, CHUNK), jnp.int32),
            pltpu.VMEM((CPW, CHUNK), jnp.int32),
            pltpu.VMEM((NB, CHUNK, WORD_DIM), jnp.float32),
            pltpu.VMEM((NB, CHUNK, POS_DIM), jnp.float32),
            pltpu.VMEM((NB, CHUNK, POS_DIM), jnp.float32),
        ]
        + [pltpu.SemaphoreType.DMA] * NB
        + [pltpu.SemaphoreType.DMA] * NB,
    )
    def k(wt, p1t, p2t, wi_h, p1i_h, p2i_h, out,
          widx_v, p1idx_v, p2idx_v, w_v, p1_v, p2_v, *sems):
        sem_g = sems[:NB]
        sem_s = sems[NB:]
        wid = lax.axis_index("s") * NC + lax.axis_index("c")
        cbase = wid * CPW
        pltpu.sync_copy(wi_h.at[wid], widx_v)
        pltpu.sync_copy(p1i_h.at[wid], p1idx_v)
        pltpu.sync_copy(p2i_h.at[wid], p2idx_v)

        def start_gather(j, b):
            pltpu.async_copy(wt.at[widx_v.at[j]], w_v.at[b], sem_g[b])
            pltpu.async_copy(p1t.at[p1idx_v.at[j]], p1_v.at[b], sem_g[b])
            pltpu.async_copy(p2t.at[p2idx_v.at[j]], p2_v.at[b], sem_g[b])

        def wait_gather(j, b):
            pltpu.make_async_copy(wt.at[widx_v.at[j]], w_v.at[b], sem_g[b]).wait()
            pltpu.make_async_copy(p1t.at[p1idx_v.at[j]], p1_v.at[b], sem_g[b]).wait()
            pltpu.make_async_copy(p2t.at[p2idx_v.at[j]], p2_v.at[b], sem_g[b]).wait()

        def out_slices(j):
            row = (cbase + j) * CHUNK
            return (
                out.at[pl.ds(row, CHUNK), pl.ds(0, WORD_DIM)],
                out.at[pl.ds(row, CHUNK), pl.ds(WORD_DIM, POS_DIM)],
                out.at[pl.ds(row, CHUNK), pl.ds(WORD_DIM + POS_DIM, POS_DIM)],
            )

        def start_scatter(j, b):
            ow, o1, o2 = out_slices(j)
            pltpu.async_copy(w_v.at[b], ow, sem_s[b])
            pltpu.async_copy(p1_v.at[b], o1, sem_s[b])
            pltpu.async_copy(p2_v.at[b], o2, sem_s[b])

        def wait_scatter(j, b):
            ow, o1, o2 = out_slices(j)
            pltpu.make_async_copy(w_v.at[b], ow, sem_s[b]).wait()
            pltpu.make_async_copy(p1_v.at[b], o1, sem_s[b]).wait()
            pltpu.make_async_copy(p2_v.at[b], o2, sem_s[b]).wait()

        for b in range(NB - 1):
            start_gather(b, b)

        def step(i, carry):
            for b in range(NB):
                j = i * NB + b
                if b == 0:
                    @pl.when(j >= 1)
                    def _():
                        wait_scatter(j - 1, NB - 1)
                else:
                    wait_scatter(j - 1, b - 1)

                @pl.when(j + NB - 1 < CPW)
                def _():
                    start_gather(j + NB - 1, (b + NB - 1) % NB)

                wait_gather(j, b)
                start_scatter(j, b)
            return carry

        lax.fori_loop(0, NSTEP, step, 0)
        wait_scatter(CPW - 1, NB - 1)

    return k(word_table, pos1_table, pos2_table, wi, p1i, p2i)


def kernel(word_table, pos1_table, pos2_table, word, pos1, pos2):
    wi = word.reshape(N).astype(jnp.int32).reshape(NW, CPW, CHUNK)
    p1i = pos1.reshape(N).astype(jnp.int32).reshape(NW, CPW, CHUNK)
    p2i = pos2.reshape(N).astype(jnp.int32).reshape(NW, CPW, CHUNK)
    out = _lookup(word_table, pos1_table, pos2_table, wi, p1i, p2i)
    return out.reshape(B, L, OUT_D)
